# Initial kernel scaffold; baseline (speedup 1.0000x reference)
import jax
import jax.numpy as jnp
from jax.experimental import pallas as pl


def kernel(x, edge_index, W1_l, b1, W1_r, W2_l, b2, W2_r):
    raise NotImplementedError("write your pallas kernel here")


# speedup vs baseline: 3.8585x; 3.8585x over previous
import functools

import jax
import jax.numpy as jnp
from jax import lax
from jax.experimental import pallas as pl
from jax.experimental.pallas import tpu as pltpu
from jax.experimental.pallas import tpu_sc as plsc

N = 10000
E = 320000
D = 128
H = 128
C = 40
CP = 48
DH = 64

NC = 2
NS = 16
K = 80
EPS = E // NS
NCH1 = EPS // K
EPW = E // (NC * NS)
NCH2 = EPW // K
NP = 10240
RPT = NP // NS

BM = 400
GRID = N // BM

_mesh = plsc.VectorSubcoreMesh(core_axis_name="c", subcore_axis_name="s")


@functools.partial(
    pl.kernel, mesh=_mesh,
    compiler_params=pltpu.CompilerParams(use_tc_tiling_on_sc=False),
    out_type=[jax.ShapeDtypeStruct((NC, NP, DH), jnp.float32),
              jax.ShapeDtypeStruct((NP, 16), jnp.float32)],
    scratch_types=[
        pltpu.VMEM((K,), jnp.int32),
        pltpu.VMEM((K,), jnp.int32),
        pltpu.VMEM((K, DH), jnp.float32),
        pltpu.VMEM((RPT, DH), jnp.float32),
        pltpu.VMEM_SHARED((NP, DH), jnp.float32),
        pltpu.SemaphoreType.DMA,
        pltpu.VMEM((K, 16), jnp.float32),
        pltpu.VMEM((RPT, 16), jnp.float32),
        pltpu.VMEM_SHARED((NP, 16), jnp.float32),
    ])
def _seg_sum_l1(pa_hbm, pb_hbm, src_hbm, dst_hbm, zrow_hbm, z16_hbm, ones_hbm,
                out_hbm, deg_hbm, src_v, dst_v, rows_v, zbuf, acc, sem,
                ones_v, zbuf16, dacc):
  c = lax.axis_index("c")
  s = lax.axis_index("s")

  pltpu.sync_copy(zrow_hbm, zbuf)
  pltpu.sync_copy(zbuf, acc.at[pl.ds(s * RPT, RPT)])
  pltpu.sync_copy(ones_hbm, ones_v)
  pltpu.sync_copy(z16_hbm, zbuf16)
  pltpu.sync_copy(zbuf16, dacc.at[pl.ds(s * RPT, RPT)])
  plsc.subcore_barrier()

  def body(i, _):
    base = s * EPS + i * K
    pltpu.sync_copy(src_hbm.at[pl.ds(base, K)], src_v)
    pltpu.sync_copy(dst_hbm.at[pl.ds(base, K)], dst_v)

    @pl.when(c == 0)
    def _():
      pltpu.async_copy(pa_hbm.at[src_v], rows_v, sem).wait()

    @pl.when(c == 1)
    def _():
      pltpu.async_copy(pb_hbm.at[src_v], rows_v, sem).wait()

    pltpu.sync_copy(rows_v, acc.at[dst_v], add=True)

    @pl.when(c == 0)
    def _():
      pltpu.sync_copy(ones_v, dacc.at[dst_v], add=True)

    return _

  lax.fori_loop(0, NCH1, body, None)
  plsc.subcore_barrier()

  pltpu.sync_copy(acc.at[pl.ds(s * RPT, RPT)], zbuf)
  pltpu.sync_copy(zbuf, out_hbm.at[c, pl.ds(s * RPT, RPT)])

  @pl.when(c == 0)
  def _():
    pltpu.sync_copy(dacc.at[pl.ds(s * RPT, RPT)], zbuf16)
    pltpu.sync_copy(zbuf16, deg_hbm.at[pl.ds(s * RPT, RPT)])


@functools.partial(
    pl.kernel, mesh=_mesh,
    compiler_params=pltpu.CompilerParams(use_tc_tiling_on_sc=False),
    out_type=[jax.ShapeDtypeStruct((NC, NP, CP), jnp.float32)],
    scratch_types=[
        pltpu.VMEM((K,), jnp.int32),
        pltpu.VMEM((K,), jnp.int32),
        pltpu.VMEM((K, CP), jnp.float32),
        pltpu.VMEM((RPT, CP), jnp.float32),
        pltpu.VMEM_SHARED((NP, CP), jnp.float32),
        pltpu.SemaphoreType.DMA,
    ])
def _seg_sum_l2(p_hbm, src_hbm, dst_hbm, zrow_hbm,
                out_hbm, src_v, dst_v, rows_v, zbuf, acc, sem):
  c = lax.axis_index("c")
  s = lax.axis_index("s")
  wid = c * NS + s

  pltpu.sync_copy(zrow_hbm, zbuf)
  pltpu.sync_copy(zbuf, acc.at[pl.ds(s * RPT, RPT)])
  plsc.subcore_barrier()

  def body(i, _):
    base = wid * EPW + i * K
    pltpu.sync_copy(src_hbm.at[pl.ds(base, K)], src_v)
    pltpu.sync_copy(dst_hbm.at[pl.ds(base, K)], dst_v)
    pltpu.async_copy(p_hbm.at[src_v], rows_v, sem).wait()
    pltpu.sync_copy(rows_v, acc.at[dst_v], add=True)
    return _

  lax.fori_loop(0, NCH2, body, None)
  plsc.subcore_barrier()

  pltpu.sync_copy(acc.at[pl.ds(s * RPT, RPT)], zbuf)
  pltpu.sync_copy(zbuf, out_hbm.at[c, pl.ds(s * RPT, RPT)])


def _proj_body(x_ref, w_ref, oa_ref, ob_ref):
  p = jnp.dot(x_ref[...], w_ref[...], preferred_element_type=jnp.float32)
  oa_ref[...] = p[:, :DH]
  ob_ref[...] = p[:, DH:]


def _combine1_body(x_ref, a0_ref, a1_ref, d_ref, w1r_ref, b1_ref,
                   w2l_ref, h_ref, p2_ref):
  rdeg = 1.0 / jnp.maximum(d_ref[:, 0:1], 1.0)
  mean = jnp.concatenate([a0_ref[...], a1_ref[...]], axis=1) * rdeg
  h = jnp.maximum(
      mean + jnp.dot(x_ref[...], w1r_ref[...],
                     preferred_element_type=jnp.float32) + b1_ref[...], 0.0)
  h_ref[...] = h
  p2_ref[...] = jnp.dot(h, w2l_ref[...], preferred_element_type=jnp.float32)


def _combine2_body(h_ref, a0_ref, a1_ref, d_ref, w2r_ref, b2_ref, o_ref):
  rdeg = 1.0 / jnp.maximum(d_ref[:, 0:1], 1.0)
  mean = (a0_ref[...] + a1_ref[...]) * rdeg
  o_ref[...] = mean + jnp.dot(h_ref[...], w2r_ref[...],
                              preferred_element_type=jnp.float32) + b2_ref[...]


def _row_spec(w):
  return pl.BlockSpec((BM, w), lambda i: (i, 0))


def _full_spec(r, c_):
  return pl.BlockSpec((r, c_), lambda i: (0, 0))


_proj = pl.pallas_call(
    _proj_body,
    grid=(GRID,),
    in_specs=[_row_spec(D), _full_spec(D, H)],
    out_specs=[_row_spec(DH), _row_spec(DH)],
    out_shape=[jax.ShapeDtypeStruct((N, DH), jnp.float32),
               jax.ShapeDtypeStruct((N, DH), jnp.float32)],
)

_combine1 = pl.pallas_call(
    _combine1_body,
    grid=(GRID,),
    in_specs=[_row_spec(D), _row_spec(DH), _row_spec(DH), _row_spec(16),
              _full_spec(D, H), _full_spec(1, H), _full_spec(H, CP)],
    out_specs=[_row_spec(H), _row_spec(CP)],
    out_shape=[jax.ShapeDtypeStruct((N, H), jnp.float32),
               jax.ShapeDtypeStruct((N, CP), jnp.float32)],
)

_combine2 = pl.pallas_call(
    _combine2_body,
    grid=(GRID,),
    in_specs=[_row_spec(H), _row_spec(CP), _row_spec(CP), _row_spec(16),
              _full_spec(H, CP), _full_spec(1, CP)],
    out_specs=_row_spec(CP),
    out_shape=jax.ShapeDtypeStruct((N, CP), jnp.float32),
)


@jax.jit
def kernel(x, edge_index, W1_l, b1, W1_r, W2_l, b2, W2_r):
  src = edge_index[0]
  dst = edge_index[1]

  zrow = jnp.zeros((RPT, DH), jnp.float32)
  z16 = jnp.zeros((RPT, 16), jnp.float32)
  ones = jnp.ones((K, 16), jnp.float32)
  zrow2 = jnp.zeros((RPT, CP), jnp.float32)

  w2l_p = jnp.pad(W2_l, ((0, 0), (0, CP - C)))
  w2r_p = jnp.pad(W2_r, ((0, 0), (0, CP - C)))
  b2_p = jnp.pad(b2, (0, CP - C)).reshape(1, CP)
  b1_r = b1.reshape(1, H)

  p1a, p1b = _proj(x, W1_l)
  agg1, deg = _seg_sum_l1(p1a, p1b, src, dst, zrow, z16, ones)
  h, p2 = _combine1(x, agg1[0], agg1[1], deg, W1_r, b1_r, w2l_p)
  agg2, = _seg_sum_l2(p2, src, dst, zrow2)
  out = _combine2(h, agg2[0], agg2[1], deg, w2r_p, b2_p)
  return out[:, :C]
